# baseline (speedup 1.0000x reference)
import functools

import numpy as np
import jax
import jax.numpy as jnp
from jax import lax
from jax.experimental import pallas as pl
from jax.experimental.pallas import tpu as pltpu
from jax.experimental.pallas import tpu_sc as plsc

_B, _W, _N, _F, _H = 8, 8, 64, 32, 64
_E = _N * (_N - 1)
_L = 16
_R, _C = 72, 128

_ii, _jj = np.meshgrid(np.arange(_N), np.arange(_N), indexing="ij")
_m = _ii != _jj
_PR_NP = _ii[_m].astype(np.int32)
_PC_NP = _jj[_m].astype(np.int32)
_PRPC_NP = np.concatenate([_PR_NP, _PC_NP])

_CONTRACT_T_LHS = (((0,), (0,)), ((), ()))
_CONTRACT_T_RHS = (((1,), (1,)), ((), ()))


def _tc_body(x_ref, wgcn_ref, bgcn_ref, we1t_ref, be1_ref, we2_ref, be2_ref,
             wew_ref, bew_ref, wpt_ref, bp_ref, ewf_ref, x_v, sem):
    cp_x = pltpu.make_async_copy(x_ref.at[:, _W - 1], x_v, sem)
    cp_x.start()
    cp_x.wait()
    wgcn = wgcn_ref[...]
    we1t_a = we1t_ref[:, 0:_H]
    we1t_c = we1t_ref[:, _H:2 * _H]
    wpt = wpt_ref[...]
    wq = lax.dot_general(wew_ref[...], wpt, _CONTRACT_T_RHS,
                         preferred_element_type=jnp.float32)
    cq = lax.dot_general(bew_ref[...], wpt, _CONTRACT_T_RHS,
                         preferred_element_type=jnp.float32)
    we2 = we2_ref[...]
    be2 = be2_ref[0, 0]
    ii = lax.broadcasted_iota(jnp.int32, (_N, _N), 0)
    jj = lax.broadcasted_iota(jnp.int32, (_N, _N), 1)
    diag = ii == jj
    bp_row = jnp.broadcast_to(bp_ref[...], (1, _N))
    for b in range(_B):
        ne_b = lax.dot_general(x_v[b], wgcn, _CONTRACT_T_LHS,
                               preferred_element_type=jnp.float32)
        ne_b = ne_b + bgcn_ref[...]
        Ab = lax.dot_general(ne_b, we1t_a, _CONTRACT_T_RHS,
                             preferred_element_type=jnp.float32)
        Ab = Ab + be1_ref[...]
        Cb = lax.dot_general(ne_b, we1t_c, _CONTRACT_T_RHS,
                             preferred_element_type=jnp.float32)
        qb = jnp.dot(ne_b, wq, preferred_element_type=jnp.float32) + cq
        T = jnp.maximum(Ab[:, None, :] + Cb[None, :, :], 0.0)
        ewb = jnp.sum(T * we2[None, :, :], axis=2) + be2
        ewf_ref[b, 0:_N, 0:_N] = jnp.where(diag, qb, ewb)
        ewf_ref[b, _N:_N + 1, 0:_N] = bp_row


_tc_call = pl.pallas_call(
    _tc_body,
    grid=(1,),
    in_specs=[pl.BlockSpec(memory_space=pl.ANY)]
    + [pl.BlockSpec(s, lambda i, _r=len(s): (0,) * _r)
       for s in [(_F, _H), (1, _H), (_H, 2 * _H), (1, _H), (1, _H), (1, 1),
                 (_H, _H), (1, _H), (1, _H), (1, 1)]],
    out_specs=pl.BlockSpec((_B, _R, _C), lambda i: (0, 0, 0)),
    out_shape=jax.ShapeDtypeStruct((_B, _R, _C), jnp.float32),
    scratch_shapes=[
        pltpu.VMEM((_B, _F, _N), jnp.float32),
        pltpu.SemaphoreType.DMA,
    ],
)

_NC, _NS = 2, 16


_EC = _E // 4


def _sc_body(ewf_hbm, idx_hbm,
             pred_hbm, ew_hbm,
             ewf_v, idx_v, ewo_v, acc_v, pred_v, tmp4_v, ewrow_v,
             shared_v, shared_ew, sem):
    c_id = lax.axis_index("c")
    s_id = lax.axis_index("s")
    b = c_id * 4 + s_id // 4
    k = s_id % 4
    e_base = k * _EC

    copies = [
        pltpu.make_async_copy(ewf_hbm.at[b], ewf_v, sem),
        pltpu.make_async_copy(idx_hbm.at[pl.ds(e_base, _EC)],
                              idx_v.at[pl.ds(0, _EC)], sem),
        pltpu.make_async_copy(idx_hbm.at[pl.ds(_E + e_base, _EC)],
                              idx_v.at[pl.ds(_EC, _EC)], sem),
        pltpu.make_async_copy(idx_hbm.at[pl.ds(2 * _E + e_base, _EC)],
                              idx_v.at[pl.ds(2 * _EC, _EC)], sem),
        pltpu.make_async_copy(idx_hbm.at[pl.ds(3 * _E + e_base, _EC)],
                              idx_v.at[pl.ds(3 * _EC, _EC)], sem),
    ]
    for c in copies:
        c.start()

    zero = jnp.zeros((_L,), jnp.float32)
    for i in range(_N):
        acc_v[pl.ds(i * _L, _L)] = zero

    for c in copies:
        c.wait()

    lane = lax.iota(jnp.int32, _L) * _N

    def body(i, carry):
        for u in range(7):
            e0 = i * (7 * _L) + u * _L
            pr = idx_v[pl.ds(2 * _EC + e0, _L)]
            pc = idx_v[pl.ds(3 * _EC + e0, _L)]
            ew16 = plsc.load_gather(ewf_v, [pr, pc])
            ewo_v[pl.ds(e0, _L)] = ew16
            si = idx_v[pl.ds(e0, _L)]
            qv = plsc.load_gather(ewf_v, [si, si])
            ti = idx_v[pl.ds(_EC + e0, _L)]
            plsc.addupdate_scatter(acc_v, [lane + ti], ew16 * qv)
        return carry

    lax.fori_loop(0, _EC // (7 * _L), body, 0)

    for sl in range(_N // _L):
        s = acc_v[pl.ds(sl * _L, _L)]
        for p in range(1, _L):
            s = s + acc_v[pl.ds(p * _N + sl * _L, _L)]
        pred_v[pl.ds(sl * _L, _L)] = s

    cp_ew = pltpu.make_async_copy(
        ewo_v, shared_ew.at[pl.ds((s_id // 4) * _E + e_base, _EC)], sem)
    cp_ew.start()
    pltpu.sync_copy(pred_v, shared_v.at[pl.ds(s_id * _N, _N)])
    cp_ew.wait()
    plsc.subcore_barrier()

    @pl.when(k == 0)
    def _():
        pltpu.sync_copy(shared_v.at[pl.ds(s_id * _N, 4 * _N)], tmp4_v)
        bp = ewf_v[_N, pl.ds(0, _L)]
        for sl in range(_N // _L):
            s = bp
            for r in range(4):
                s = s + tmp4_v[pl.ds(r * _N + sl * _L, _L)]
            pred_v[pl.ds(sl * _L, _L)] = s
        pltpu.sync_copy(pred_v, pred_hbm.at[b])
        pltpu.sync_copy(shared_ew.at[pl.ds((s_id // 4) * _E, _E)], ewrow_v)
        pltpu.sync_copy(ewrow_v, ew_hbm.at[b])


@functools.cache
def _make_sc_call():
    mesh = plsc.VectorSubcoreMesh(
        core_axis_name="c", subcore_axis_name="s",
        num_cores=_NC, num_subcores=_NS)
    return pl.kernel(
        _sc_body,
        out_type=[
            jax.ShapeDtypeStruct((_B, _N), jnp.float32),
            jax.ShapeDtypeStruct((_B, _E), jnp.float32),
        ],
        mesh=mesh,
        compiler_params=pltpu.CompilerParams(needs_layout_passes=False),
        scratch_types=[
            pltpu.VMEM((_R, _C), jnp.float32),
            pltpu.VMEM((4 * _EC,), jnp.int32),
            pltpu.VMEM((_EC,), jnp.float32),
            pltpu.VMEM((_L * _N,), jnp.float32),
            pltpu.VMEM((_N,), jnp.float32),
            pltpu.VMEM((4 * _N,), jnp.float32),
            pltpu.VMEM((_E,), jnp.float32),
            pltpu.VMEM_SHARED((_NS * _N,), jnp.float32),
            pltpu.VMEM_SHARED((4 * _E,), jnp.float32),
            pltpu.SemaphoreType.DMA,
        ],
    )


def kernel(x_seq, edge_index, W_gcn, b_gcn, W_e1, b_e1, W_e2, b_e2,
           W_ew, b_ew, W_ih, W_hh, b_ih, b_hh, W_p, b_p):
    x_t = jnp.swapaxes(x_seq, 2, 3)
    we1_t = W_e1.T
    wp_t = W_p.T
    ewf = _tc_call(
        x_t, W_gcn, b_gcn.reshape(1, _H), we1_t, b_e1.reshape(1, _H),
        W_e2.reshape(1, _H), b_e2.reshape(1, 1), W_ew, b_ew.reshape(1, _H),
        wp_t, b_p.reshape(1, 1))
    idx = jnp.concatenate(
        [edge_index[0], edge_index[1], jnp.asarray(_PRPC_NP)])
    pred, ew_out = _make_sc_call()(ewf, idx)
    return pred, ew_out


# speedup vs baseline: 1.0015x; 1.0015x over previous
import functools

import numpy as np
import jax
import jax.numpy as jnp
from jax import lax
from jax.experimental import pallas as pl
from jax.experimental.pallas import tpu as pltpu
from jax.experimental.pallas import tpu_sc as plsc

_B, _W, _N, _F, _H = 8, 8, 64, 32, 64
_E = _N * (_N - 1)
_L = 16
_R, _C = 72, 128

_ii, _jj = np.meshgrid(np.arange(_N), np.arange(_N), indexing="ij")
_m = _ii != _jj
_PR_NP = _ii[_m].astype(np.int32)
_PC_NP = _jj[_m].astype(np.int32)
_PRPC_NP = np.concatenate([_PR_NP, _PC_NP])

_CONTRACT_T_LHS = (((0,), (0,)), ((), ()))
_CONTRACT_T_RHS = (((1,), (1,)), ((), ()))


def _tc_body(x_ref, wgcn_ref, bgcn_ref, we1t_ref, be1_ref, we2_ref, be2_ref,
             wew_ref, bew_ref, wpt_ref, bp_ref, ewf_ref, x_v, sem):
    cp_x = pltpu.make_async_copy(x_ref.at[:, _W - 1], x_v, sem)
    cp_x.start()
    cp_x.wait()
    wgcn = wgcn_ref[...]
    we1t_a = we1t_ref[:, 0:_H]
    we1t_c = we1t_ref[:, _H:2 * _H]
    wpt = wpt_ref[...]
    wq = lax.dot_general(wew_ref[...], wpt, _CONTRACT_T_RHS,
                         preferred_element_type=jnp.float32)
    cq = lax.dot_general(bew_ref[...], wpt, _CONTRACT_T_RHS,
                         preferred_element_type=jnp.float32)
    we2 = we2_ref[...]
    be2 = be2_ref[0, 0]
    ii = lax.broadcasted_iota(jnp.int32, (_N, _N), 0)
    jj = lax.broadcasted_iota(jnp.int32, (_N, _N), 1)
    diag = ii == jj
    bp_row = jnp.broadcast_to(bp_ref[...], (1, _N))
    for b in range(_B):
        ne_b = lax.dot_general(x_v[b], wgcn, _CONTRACT_T_LHS,
                               preferred_element_type=jnp.float32)
        ne_b = ne_b + bgcn_ref[...]
        Ab = lax.dot_general(ne_b, we1t_a, _CONTRACT_T_RHS,
                             preferred_element_type=jnp.float32)
        Ab = Ab + be1_ref[...]
        Cb = lax.dot_general(ne_b, we1t_c, _CONTRACT_T_RHS,
                             preferred_element_type=jnp.float32)
        qb = jnp.dot(ne_b, wq, preferred_element_type=jnp.float32) + cq
        T = jnp.maximum(Ab[:, None, :] + Cb[None, :, :], 0.0)
        ewb = jnp.sum(T * we2[None, :, :], axis=2) + be2
        ewf_ref[b, 0:_N, 0:_N] = jnp.where(diag, qb, ewb)
        ewf_ref[b, _N:_N + 1, 0:_N] = bp_row


_tc_call = pl.pallas_call(
    _tc_body,
    grid=(1,),
    in_specs=[pl.BlockSpec(memory_space=pl.ANY)]
    + [pl.BlockSpec(s, lambda i, _r=len(s): (0,) * _r)
       for s in [(_F, _H), (1, _H), (_H, 2 * _H), (1, _H), (1, _H), (1, 1),
                 (_H, _H), (1, _H), (1, _H), (1, 1)]],
    out_specs=pl.BlockSpec((_B, _R, _C), lambda i: (0, 0, 0)),
    out_shape=jax.ShapeDtypeStruct((_B, _R, _C), jnp.float32),
    scratch_shapes=[
        pltpu.VMEM((_B, _F, _N), jnp.float32),
        pltpu.SemaphoreType.DMA,
    ],
)

_NC, _NS = 2, 16


_EC = _E // 4


def _sc_body(ewf_hbm, idx_hbm,
             pred_hbm, ew_hbm,
             ewf_v, idx_v, ewo_v, acc_v, pred_v, tmp4_v, ewrow_v,
             shared_v, shared_ew, sem):
    c_id = lax.axis_index("c")
    s_id = lax.axis_index("s")
    b = c_id * 4 + s_id // 4
    k = s_id % 4
    e_base = k * _EC

    copies = [
        pltpu.make_async_copy(ewf_hbm.at[b], ewf_v, sem),
        pltpu.make_async_copy(idx_hbm.at[pl.ds(e_base, _EC)],
                              idx_v.at[pl.ds(0, _EC)], sem),
        pltpu.make_async_copy(idx_hbm.at[pl.ds(_E + e_base, _EC)],
                              idx_v.at[pl.ds(_EC, _EC)], sem),
        pltpu.make_async_copy(idx_hbm.at[pl.ds(2 * _E + e_base, _EC)],
                              idx_v.at[pl.ds(2 * _EC, _EC)], sem),
        pltpu.make_async_copy(idx_hbm.at[pl.ds(3 * _E + e_base, _EC)],
                              idx_v.at[pl.ds(3 * _EC, _EC)], sem),
    ]
    for c in copies:
        c.start()

    zero = jnp.zeros((_L,), jnp.float32)
    for i in range(_N):
        acc_v[pl.ds(i * _L, _L)] = zero

    for c in copies:
        c.wait()

    lane = lax.iota(jnp.int32, _L) * _N

    def body(i, carry):
        for u in range(3):
            e0 = i * (3 * _L) + u * _L
            pr = idx_v[pl.ds(2 * _EC + e0, _L)]
            pc = idx_v[pl.ds(3 * _EC + e0, _L)]
            ew16 = plsc.load_gather(ewf_v, [pr, pc])
            ewo_v[pl.ds(e0, _L)] = ew16
            si = idx_v[pl.ds(e0, _L)]
            qv = plsc.load_gather(ewf_v, [si, si])
            ti = idx_v[pl.ds(_EC + e0, _L)]
            plsc.addupdate_scatter(acc_v, [lane + ti], ew16 * qv)
        return carry

    lax.fori_loop(0, _EC // (3 * _L), body, 0)

    for sl in range(_N // _L):
        s = acc_v[pl.ds(sl * _L, _L)]
        for p in range(1, _L):
            s = s + acc_v[pl.ds(p * _N + sl * _L, _L)]
        pred_v[pl.ds(sl * _L, _L)] = s

    cp_ew = pltpu.make_async_copy(
        ewo_v, shared_ew.at[pl.ds((s_id // 4) * _E + e_base, _EC)], sem)
    cp_ew.start()
    pltpu.sync_copy(pred_v, shared_v.at[pl.ds(s_id * _N, _N)])
    cp_ew.wait()
    plsc.subcore_barrier()

    @pl.when(k == 0)
    def _():
        pltpu.sync_copy(shared_v.at[pl.ds(s_id * _N, 4 * _N)], tmp4_v)
        bp = ewf_v[_N, pl.ds(0, _L)]
        for sl in range(_N // _L):
            s = bp
            for r in range(4):
                s = s + tmp4_v[pl.ds(r * _N + sl * _L, _L)]
            pred_v[pl.ds(sl * _L, _L)] = s
        pltpu.sync_copy(pred_v, pred_hbm.at[b])
        pltpu.sync_copy(shared_ew.at[pl.ds((s_id // 4) * _E, _E)], ewrow_v)
        pltpu.sync_copy(ewrow_v, ew_hbm.at[b])


@functools.cache
def _make_sc_call():
    mesh = plsc.VectorSubcoreMesh(
        core_axis_name="c", subcore_axis_name="s",
        num_cores=_NC, num_subcores=_NS)
    return pl.kernel(
        _sc_body,
        out_type=[
            jax.ShapeDtypeStruct((_B, _N), jnp.float32),
            jax.ShapeDtypeStruct((_B, _E), jnp.float32),
        ],
        mesh=mesh,
        compiler_params=pltpu.CompilerParams(needs_layout_passes=False),
        scratch_types=[
            pltpu.VMEM((_R, _C), jnp.float32),
            pltpu.VMEM((4 * _EC,), jnp.int32),
            pltpu.VMEM((_EC,), jnp.float32),
            pltpu.VMEM((_L * _N,), jnp.float32),
            pltpu.VMEM((_N,), jnp.float32),
            pltpu.VMEM((4 * _N,), jnp.float32),
            pltpu.VMEM((_E,), jnp.float32),
            pltpu.VMEM_SHARED((_NS * _N,), jnp.float32),
            pltpu.VMEM_SHARED((4 * _E,), jnp.float32),
            pltpu.SemaphoreType.DMA,
        ],
    )


def kernel(x_seq, edge_index, W_gcn, b_gcn, W_e1, b_e1, W_e2, b_e2,
           W_ew, b_ew, W_ih, W_hh, b_ih, b_hh, W_p, b_p):
    x_t = jnp.swapaxes(x_seq, 2, 3)
    we1_t = W_e1.T
    wp_t = W_p.T
    ewf = _tc_call(
        x_t, W_gcn, b_gcn.reshape(1, _H), we1_t, b_e1.reshape(1, _H),
        W_e2.reshape(1, _H), b_e2.reshape(1, 1), W_ew, b_ew.reshape(1, _H),
        wp_t, b_p.reshape(1, 1))
    idx = jnp.concatenate(
        [edge_index[0], edge_index[1], jnp.asarray(_PRPC_NP)])
    pred, ew_out = _make_sc_call()(ewf, idx)
    return pred, ew_out
